# baseline (speedup 1.0000x reference)
import jax
import jax.numpy as jnp
from jax import lax
from jax.experimental import pallas as pl
from jax.experimental.pallas import tpu as pltpu
from jax.experimental.pallas import tpu_sc as plsc

B = 4096
L = 200
TOK = B * L
LOC_EMB = 32
TIME_EMB = 32
MODE_EMB = 16
MODE_VOC = 8
CMB_EMB = TIME_EMB + MODE_EMB
OUT_D = LOC_EMB + CMB_EMB

IDXW = 128
LANES = 16
NW = 32
TOK_PER_W = TOK // NW
CHUNK = 640
N_CHUNKS = TOK_PER_W // CHUNK
G_PER_CHUNK = CHUNK // IDXW
V_PER_CHUNK = CHUNK // LANES


def _body(src_hbm, time_hbm, mode_hbm, loc_tab, cmb_tab, out_hbm,
          sidx0, sidx1, tidx0, tidx1, midx0, midx1, fidx0, fidx1,
          loc0, loc1, cmb0, cmb1,
          isem0, isem1, gsem0, gsem1, wsem0, wsem1):
    cid = lax.axis_index("c")
    sid = lax.axis_index("s")
    wid = sid * 2 + cid
    tbase = wid * TOK_PER_W

    bufs = ((sidx0, tidx0, midx0, fidx0, loc0, cmb0, isem0, gsem0, wsem0),
            (sidx1, tidx1, midx1, fidx1, loc1, cmb1, isem1, gsem1, wsem1))

    def fire_idx(g, p):
        sidx, tidx, midx, fidx, loc, cmb, isem, gsem, wsem = bufs[p]
        tok0 = tbase + g * CHUNK
        pltpu.async_copy(src_hbm.at[pl.ds(tok0, CHUNK)], sidx, isem)
        pltpu.async_copy(time_hbm.at[pl.ds(tok0, CHUNK)], tidx, isem)
        pltpu.async_copy(mode_hbm.at[pl.ds(tok0, CHUNK)], midx, isem)

    def wait_idx(p):
        sidx, tidx, midx, fidx, loc, cmb, isem, gsem, wsem = bufs[p]
        pltpu.make_async_copy(src_hbm.at[pl.ds(0, CHUNK)], sidx, isem).wait()
        pltpu.make_async_copy(time_hbm.at[pl.ds(0, CHUNK)], tidx, isem).wait()
        pltpu.make_async_copy(mode_hbm.at[pl.ds(0, CHUNK)], midx, isem).wait()

    def fuse(p):
        sidx, tidx, midx, fidx, loc, cmb, isem, gsem, wsem = bufs[p]

        def step(v, c2):
            o = v * LANES
            fidx[pl.ds(o, LANES)] = tidx[pl.ds(o, LANES)] * MODE_VOC + midx[pl.ds(o, LANES)]
            return c2

        lax.fori_loop(0, V_PER_CHUNK, step, 0)

    def fire_gathers(p):
        sidx, tidx, midx, fidx, loc, cmb, isem, gsem, wsem = bufs[p]
        for j in range(G_PER_CHUNK):
            o = j * IDXW
            pltpu.async_copy(loc_tab.at[sidx.at[pl.ds(o, IDXW)]],
                             loc.at[pl.ds(o, IDXW)], gsem)
            pltpu.async_copy(cmb_tab.at[fidx.at[pl.ds(o, IDXW)]],
                             cmb.at[pl.ds(o, IDXW)], gsem)

    def wait_gathers(p):
        sidx, tidx, midx, fidx, loc, cmb, isem, gsem, wsem = bufs[p]
        pltpu.make_async_copy(loc_tab.at[sidx], loc, gsem).wait()
        pltpu.make_async_copy(cmb_tab.at[fidx], cmb, gsem).wait()

    def fire_writes(g, p):
        sidx, tidx, midx, fidx, loc, cmb, isem, gsem, wsem = bufs[p]
        tok0 = tbase + g * CHUNK
        pltpu.async_copy(loc, out_hbm.at[pl.ds(tok0, CHUNK), pl.ds(0, LOC_EMB)], wsem)
        pltpu.async_copy(cmb, out_hbm.at[pl.ds(tok0, CHUNK), pl.ds(LOC_EMB, CMB_EMB)], wsem)

    def wait_writes(p):
        sidx, tidx, midx, fidx, loc, cmb, isem, gsem, wsem = bufs[p]
        pltpu.make_async_copy(loc, out_hbm.at[pl.ds(0, CHUNK), pl.ds(0, LOC_EMB)], wsem).wait()
        pltpu.make_async_copy(cmb, out_hbm.at[pl.ds(0, CHUNK), pl.ds(LOC_EMB, CMB_EMB)], wsem).wait()

    fire_idx(0, 0)
    fire_idx(1, 1)
    wait_idx(0)
    fuse(0)
    fire_gathers(0)

    def chunk_step(g, carry):
        def on_parity(p):
            def go():
                wait_idx(p)
                fuse(p)

                @pl.when(g >= 2)
                def _():
                    wait_writes(p)

                fire_gathers(p)
                wait_gathers(1 - p)
                fire_writes(g - 1, 1 - p)

                @pl.when(g < N_CHUNKS - 1)
                def _():
                    fire_idx(g + 1, 1 - p)
            return go

        @pl.when(g % 2 == 0)
        def _():
            on_parity(0)()

        @pl.when(g % 2 == 1)
        def _():
            on_parity(1)()
        return carry

    lax.fori_loop(1, N_CHUNKS, chunk_step, 0)

    pl_last = (N_CHUNKS - 1) % 2
    wait_gathers(pl_last)
    fire_writes(N_CHUNKS - 1, pl_last)
    wait_writes(1 - pl_last)
    wait_writes(pl_last)


@jax.jit
def _run(src1d, time1d, mode1d, loc_tab, cmb_tab):
    mesh = plsc.VectorSubcoreMesh(core_axis_name="c", subcore_axis_name="s")
    idx_t = pltpu.VMEM((CHUNK,), jnp.int32)
    k = pl.kernel(
        _body,
        out_type=jax.ShapeDtypeStruct((TOK, OUT_D), jnp.float32),
        mesh=mesh,
        scratch_types=[
            idx_t, idx_t, idx_t, idx_t, idx_t, idx_t, idx_t, idx_t,
            pltpu.VMEM((CHUNK, LOC_EMB), jnp.float32),
            pltpu.VMEM((CHUNK, LOC_EMB), jnp.float32),
            pltpu.VMEM((CHUNK, CMB_EMB), jnp.float32),
            pltpu.VMEM((CHUNK, CMB_EMB), jnp.float32),
            pltpu.SemaphoreType.DMA, pltpu.SemaphoreType.DMA,
            pltpu.SemaphoreType.DMA, pltpu.SemaphoreType.DMA,
            pltpu.SemaphoreType.DMA, pltpu.SemaphoreType.DMA,
        ],
        compiler_params=pltpu.CompilerParams(use_tc_tiling_on_sc=False),
    )
    return k(src1d, time1d, mode1d, loc_tab, cmb_tab)


def kernel(src, time, mode, emb_loc_table, emb_time_table, emb_mode_table):
    cmb_tab = jnp.concatenate(
        [jnp.repeat(emb_time_table, MODE_VOC, axis=0),
         jnp.tile(emb_mode_table, (emb_time_table.shape[0], 1))], axis=-1)
    out = _run(src.astype(jnp.int32).reshape(TOK),
               time.astype(jnp.int32).reshape(TOK),
               mode.astype(jnp.int32).reshape(TOK),
               emb_loc_table, cmb_tab)
    return out.reshape(B, L, OUT_D)
